# baseline (speedup 1.0000x reference)
import functools

import jax
import jax.numpy as jnp
from jax import lax
from jax.experimental import pallas as pl
from jax.experimental.pallas import tpu as pltpu
from jax.experimental.pallas import tpu_sc as plsc

_EPS = 1e-5
_BIG = 3e38
_K = 16
_NC, _NS = 2, 16
_NW = _NC * _NS
_GW = 128


def _topk_body(nq_blk, n_cand, xT_ref, qT_ref, idx_ref):
    b = pl.program_id(0)
    xT = xT_ref[0]
    qT = qT_ref[0]
    dot = lax.dot_general(qT, xT, (((0,), (0,)), ((), ())),
                          preferred_element_type=jnp.float32)
    x2 = jnp.sum(xT * xT, axis=0)
    dmat = x2[None, :] - (dot + dot)
    iota = lax.broadcasted_iota(jnp.int32, (nq_blk, n_cand), 1)
    cols = []
    for _ in range(_K):
        first = jnp.argmin(dmat, axis=1).astype(jnp.int32)[:, None]
        cols.append(first)
        dmat = jnp.where(iota == first, _BIG, dmat)
    idx_ref[0] = jnp.concatenate(cols, axis=1) + b * n_cand


def _topk(xT, qT, nq_blk=512):
    B, _, n = xT.shape
    M = qT.shape[2]
    return pl.pallas_call(
        functools.partial(_topk_body, nq_blk, n),
        grid=(B, M // nq_blk),
        in_specs=[
            pl.BlockSpec((1, 3, n), lambda b, i: (b, 0, 0)),
            pl.BlockSpec((1, 3, nq_blk), lambda b, i: (b, 0, i)),
        ],
        out_specs=pl.BlockSpec((1, nq_blk, _K), lambda b, i: (b, i, 0)),
        out_shape=jax.ShapeDtypeStruct((B, M, _K), jnp.int32),
    )(xT, qT)


def _proj_body(M, faug_ref, wa_ref, wd_ref, g_ref, cq_ref):
    fa = faug_ref[0]
    g_ref[0] = jnp.dot(fa, wa_ref[...], preferred_element_type=jnp.float32)
    cq_ref[0] = jnp.dot(fa[:M], wd_ref[...],
                        preferred_element_type=jnp.float32)


def _proj(faug, wa, wd, M):
    B, n, C = faug.shape
    Cg = wa.shape[1]
    Cq = wd.shape[1]
    return pl.pallas_call(
        functools.partial(_proj_body, M),
        grid=(B,),
        in_specs=[
            pl.BlockSpec((1, n, C), lambda b: (b, 0, 0)),
            pl.BlockSpec((C, Cg), lambda b: (0, 0)),
            pl.BlockSpec((C, Cq), lambda b: (0, 0)),
        ],
        out_specs=[
            pl.BlockSpec((1, n, Cg), lambda b: (b, 0, 0)),
            pl.BlockSpec((1, M, Cq), lambda b: (b, 0, 0)),
        ],
        out_shape=[
            jax.ShapeDtypeStruct((B, n, Cg), jnp.float32),
            jax.ShapeDtypeStruct((B, M, Cq), jnp.float32),
        ],
    )(faug, wa, wd)


def _fc_body(f_ref, w_ref, b_ref, o_ref):
    o_ref[0] = jnp.dot(f_ref[0], w_ref[...],
                       preferred_element_type=jnp.float32) + b_ref[...]


def _fc(f3, wfc, bfc):
    B, M, C = f3.shape
    Cfc = wfc.shape[1]
    return pl.pallas_call(
        _fc_body,
        grid=(B,),
        in_specs=[
            pl.BlockSpec((1, M, C), lambda b: (b, 0, 0)),
            pl.BlockSpec((C, Cfc), lambda b: (0, 0)),
            pl.BlockSpec((1, Cfc), lambda b: (0, 0)),
        ],
        out_specs=pl.BlockSpec((1, M, Cfc), lambda b: (b, 0, 0)),
        out_shape=jax.ShapeDtypeStruct((B, M, Cfc), jnp.float32),
    )(f3, wfc, bfc)


def _sc_gather_max(idx_flat, table, cq, gs, bb, qc):
    R, Du = cq.shape
    D = table.shape[1]
    rw = R // _NW
    nchunks = rw // qc
    ngath = (qc * _K) // _GW
    mesh = plsc.VectorSubcoreMesh(core_axis_name="c", subcore_axis_name="s")

    @functools.partial(
        pl.kernel, mesh=mesh,
        out_type=jax.ShapeDtypeStruct((R, Du), jnp.float32),
        scratch_types=[
            pltpu.VMEM((qc * _K,), jnp.int32),
            pltpu.VMEM((qc * _K, D), jnp.float32),
            pltpu.VMEM((qc, Du), jnp.float32),
            pltpu.VMEM((qc, Du), jnp.float32),
            pltpu.VMEM((Du,), jnp.float32),
            pltpu.VMEM((Du,), jnp.float32),
            pltpu.SemaphoreType.DMA,
        ],
    )
    def k(idx_hbm, table_hbm, cq_hbm, gs_hbm, bb_hbm, out_hbm,
          idx_v, rows_v, cq_v, out_v, gs_v, bb_v, sem):
        wid = lax.axis_index("s") * _NC + lax.axis_index("c")
        pltpu.sync_copy(gs_hbm, gs_v)
        pltpu.sync_copy(bb_hbm, bb_v)
        for t in range(nchunks):
            r0 = wid * rw + t * qc
            pltpu.sync_copy(idx_hbm.at[pl.ds(r0 * _K, qc * _K)], idx_v)
            handles = []
            for g in range(ngath):
                handles.append(pltpu.async_copy(
                    table_hbm.at[idx_v.at[pl.ds(g * _GW, _GW)]],
                    rows_v.at[pl.ds(g * _GW, _GW)], sem))
            pltpu.sync_copy(cq_hbm.at[pl.ds(r0, qc)], cq_v)
            for h in handles:
                h.wait()

            def qbody(q, carry):
                for c in range(Du // 16):
                    sl = pl.ds(c * 16, 16)
                    acc = rows_v[q * _K, sl]
                    for j in range(1, _K):
                        acc = jnp.maximum(acc, rows_v[q * _K + j, sl])
                    val = (acc + cq_v[q, sl]) * gs_v[sl] + bb_v[sl]
                    out_v[q, sl] = jnp.maximum(val, 0.0)
                return carry

            lax.fori_loop(0, qc, qbody, jnp.int32(0))
            pltpu.sync_copy(out_v, out_hbm.at[pl.ds(r0, qc)])

    return k(idx_flat, table, cq, gs, bb)


def kernel(pcd, W1, g1, bt1, W2, g2, bt2, W3, g3, bt3, Wfc, bfc):
    B, N, _ = pcd.shape
    M1, M2, M3 = N // 2, N // 4, N // 8
    s = (1.0 + _EPS) ** -0.5

    xyz = pcd[..., 0:3]
    xyzT = jnp.transpose(xyz, (0, 2, 1))

    def prep(W, g, b, C):
        wa = jnp.transpose(W[:, :C])
        wd = jnp.transpose(W[:, C:] - W[:, :C])
        if wa.shape[1] < 128:
            wa = jnp.pad(wa, ((0, 0), (0, 128 - wa.shape[1])))
        return wa, wd, g * s, b

    i1 = _topk(xyzT, xyzT[:, :, :M1])
    i2 = _topk(xyzT[:, :, :M1], xyzT[:, :, :M2])
    i3 = _topk(xyzT[:, :, :M2], xyzT[:, :, :M3])

    faug1 = jnp.concatenate([pcd[..., 3:6], xyz], axis=-1)
    wa1, wd1, gs1, bb1 = prep(W1, g1, bt1, 6)
    G1, CQ1 = _proj(faug1, wa1, wd1, M1)
    f1 = _sc_gather_max(i1.reshape(-1), G1.reshape(B * N, 128),
                        CQ1.reshape(B * M1, 64), gs1, bb1, qc=32)
    f1 = f1.reshape(B, M1, 64)

    faug2 = jnp.concatenate([f1, xyz[:, :M1]], axis=-1)
    wa2, wd2, gs2, bb2 = prep(W2, g2, bt2, 67)
    G2, CQ2 = _proj(faug2, wa2, wd2, M2)
    f2 = _sc_gather_max(i2.reshape(-1), G2.reshape(B * M1, 128),
                        CQ2.reshape(B * M2, 64), gs2, bb2, qc=32)
    f2 = f2.reshape(B, M2, 64)

    faug3 = jnp.concatenate([f2, xyz[:, :M2]], axis=-1)
    wa3, wd3, gs3, bb3 = prep(W3, g3, bt3, 67)
    G3, CQ3 = _proj(faug3, wa3, wd3, M3)
    f3 = _sc_gather_max(i3.reshape(-1), G3.reshape(B * M2, 128),
                        CQ3.reshape(B * M3, 128), gs3, bb3, qc=32)
    f3 = f3.reshape(B, M3, 128)
    ofc = _fc(f3, jnp.transpose(Wfc), bfc[None, :])

    x3 = xyz[:, :M3]
    out_feat = jnp.transpose(ofc, (0, 2, 1))
    f3_t = jnp.transpose(f3, (0, 2, 1))
    idx_out = jnp.broadcast_to(
        jnp.arange(M3, dtype=jnp.int64)[None, :], (B, M3)).astype(jnp.int64)
    return (x3, out_feat, idx_out, f3_t)


# speedup vs baseline: 1.0749x; 1.0749x over previous
import functools

import jax
import jax.numpy as jnp
from jax import lax
from jax.experimental import pallas as pl
from jax.experimental.pallas import tpu as pltpu
from jax.experimental.pallas import tpu_sc as plsc

_EPS = 1e-5
_BIG = 3e38
_K = 16
_NC, _NS = 2, 16
_NW = _NC * _NS
_GW = 128


def _topk_body(nq_blk, n_cand, xT_ref, qT_ref, idx_ref):
    b = pl.program_id(0)
    xT = xT_ref[0]
    qT = qT_ref[0]
    dot = lax.dot_general(qT, xT, (((0,), (0,)), ((), ())),
                          preferred_element_type=jnp.float32)
    x2 = jnp.sum(xT * xT, axis=0)
    dmat = x2[None, :] - (dot + dot)
    iota = lax.broadcasted_iota(jnp.int32, (nq_blk, n_cand), 1)
    cols = []
    for _ in range(_K):
        first = jnp.argmin(dmat, axis=1).astype(jnp.int32)[:, None]
        cols.append(first)
        dmat = jnp.where(iota == first, _BIG, dmat)
    idx_ref[0] = jnp.concatenate(cols, axis=1) + b * n_cand


def _topk(xT, qT, nq_blk=128):
    B, _, n = xT.shape
    M = qT.shape[2]
    return pl.pallas_call(
        functools.partial(_topk_body, nq_blk, n),
        grid=(B, M // nq_blk),
        in_specs=[
            pl.BlockSpec((1, 3, n), lambda b, i: (b, 0, 0)),
            pl.BlockSpec((1, 3, nq_blk), lambda b, i: (b, 0, i)),
        ],
        out_specs=pl.BlockSpec((1, nq_blk, _K), lambda b, i: (b, i, 0)),
        out_shape=jax.ShapeDtypeStruct((B, M, _K), jnp.int32),
    )(xT, qT)


def _proj_body(M, faug_ref, wa_ref, wd_ref, g_ref, cq_ref):
    fa = faug_ref[0]
    g_ref[0] = jnp.dot(fa, wa_ref[...], preferred_element_type=jnp.float32)
    cq_ref[0] = jnp.dot(fa[:M], wd_ref[...],
                        preferred_element_type=jnp.float32)


def _proj(faug, wa, wd, M):
    B, n, C = faug.shape
    Cg = wa.shape[1]
    Cq = wd.shape[1]
    return pl.pallas_call(
        functools.partial(_proj_body, M),
        grid=(B,),
        in_specs=[
            pl.BlockSpec((1, n, C), lambda b: (b, 0, 0)),
            pl.BlockSpec((C, Cg), lambda b: (0, 0)),
            pl.BlockSpec((C, Cq), lambda b: (0, 0)),
        ],
        out_specs=[
            pl.BlockSpec((1, n, Cg), lambda b: (b, 0, 0)),
            pl.BlockSpec((1, M, Cq), lambda b: (b, 0, 0)),
        ],
        out_shape=[
            jax.ShapeDtypeStruct((B, n, Cg), jnp.float32),
            jax.ShapeDtypeStruct((B, M, Cq), jnp.float32),
        ],
    )(faug, wa, wd)


def _fc_body(f_ref, w_ref, b_ref, o_ref):
    o_ref[0] = jnp.dot(f_ref[0], w_ref[...],
                       preferred_element_type=jnp.float32) + b_ref[...]


def _fc(f3, wfc, bfc):
    B, M, C = f3.shape
    Cfc = wfc.shape[1]
    return pl.pallas_call(
        _fc_body,
        grid=(B,),
        in_specs=[
            pl.BlockSpec((1, M, C), lambda b: (b, 0, 0)),
            pl.BlockSpec((C, Cfc), lambda b: (0, 0)),
            pl.BlockSpec((1, Cfc), lambda b: (0, 0)),
        ],
        out_specs=pl.BlockSpec((1, M, Cfc), lambda b: (b, 0, 0)),
        out_shape=jax.ShapeDtypeStruct((B, M, Cfc), jnp.float32),
    )(f3, wfc, bfc)


def _sc_gather_max(idx_flat, table, cq, gs, bb, qc):
    R, Du = cq.shape
    D = table.shape[1]
    rw = R // _NW
    nchunks = rw // qc
    ngath = (qc * _K) // _GW
    mesh = plsc.VectorSubcoreMesh(core_axis_name="c", subcore_axis_name="s")

    @functools.partial(
        pl.kernel, mesh=mesh,
        out_type=jax.ShapeDtypeStruct((R, Du), jnp.float32),
        scratch_types=[
            pltpu.VMEM((qc * _K,), jnp.int32),
            pltpu.VMEM((qc * _K, D), jnp.float32),
            pltpu.VMEM((qc, Du), jnp.float32),
            pltpu.VMEM((qc, Du), jnp.float32),
            pltpu.VMEM((Du,), jnp.float32),
            pltpu.VMEM((Du,), jnp.float32),
            pltpu.SemaphoreType.DMA,
        ],
    )
    def k(idx_hbm, table_hbm, cq_hbm, gs_hbm, bb_hbm, out_hbm,
          idx_v, rows_v, cq_v, out_v, gs_v, bb_v, sem):
        wid = lax.axis_index("s") * _NC + lax.axis_index("c")
        pltpu.sync_copy(gs_hbm, gs_v)
        pltpu.sync_copy(bb_hbm, bb_v)
        for t in range(nchunks):
            r0 = wid * rw + t * qc
            pltpu.sync_copy(idx_hbm.at[pl.ds(r0 * _K, qc * _K)], idx_v)
            handles = []
            for g in range(ngath):
                handles.append(pltpu.async_copy(
                    table_hbm.at[idx_v.at[pl.ds(g * _GW, _GW)]],
                    rows_v.at[pl.ds(g * _GW, _GW)], sem))
            pltpu.sync_copy(cq_hbm.at[pl.ds(r0, qc)], cq_v)
            for h in handles:
                h.wait()

            def qbody(q, carry):
                for c in range(Du // 16):
                    sl = pl.ds(c * 16, 16)
                    acc = rows_v[q * _K, sl]
                    for j in range(1, _K):
                        acc = jnp.maximum(acc, rows_v[q * _K + j, sl])
                    val = (acc + cq_v[q, sl]) * gs_v[sl] + bb_v[sl]
                    out_v[q, sl] = jnp.maximum(val, 0.0)
                return carry

            lax.fori_loop(0, qc, qbody, jnp.int32(0))
            pltpu.sync_copy(out_v, out_hbm.at[pl.ds(r0, qc)])

    return k(idx_flat, table, cq, gs, bb)


def kernel(pcd, W1, g1, bt1, W2, g2, bt2, W3, g3, bt3, Wfc, bfc):
    B, N, _ = pcd.shape
    M1, M2, M3 = N // 2, N // 4, N // 8
    s = (1.0 + _EPS) ** -0.5

    xyz = pcd[..., 0:3]
    xyzT = jnp.transpose(xyz, (0, 2, 1))

    def prep(W, g, b, C):
        wa = jnp.transpose(W[:, :C])
        wd = jnp.transpose(W[:, C:] - W[:, :C])
        if wa.shape[1] < 128:
            wa = jnp.pad(wa, ((0, 0), (0, 128 - wa.shape[1])))
        return wa, wd, g * s, b

    i1 = _topk(xyzT, xyzT[:, :, :M1])
    i2 = _topk(xyzT[:, :, :M1], xyzT[:, :, :M2])
    i3 = _topk(xyzT[:, :, :M2], xyzT[:, :, :M3])

    faug1 = jnp.concatenate([pcd[..., 3:6], xyz], axis=-1)
    wa1, wd1, gs1, bb1 = prep(W1, g1, bt1, 6)
    G1, CQ1 = _proj(faug1, wa1, wd1, M1)
    f1 = _sc_gather_max(i1.reshape(-1), G1.reshape(B * N, 128),
                        CQ1.reshape(B * M1, 64), gs1, bb1, qc=32)
    f1 = f1.reshape(B, M1, 64)

    faug2 = jnp.concatenate([f1, xyz[:, :M1]], axis=-1)
    wa2, wd2, gs2, bb2 = prep(W2, g2, bt2, 67)
    G2, CQ2 = _proj(faug2, wa2, wd2, M2)
    f2 = _sc_gather_max(i2.reshape(-1), G2.reshape(B * M1, 128),
                        CQ2.reshape(B * M2, 64), gs2, bb2, qc=32)
    f2 = f2.reshape(B, M2, 64)

    faug3 = jnp.concatenate([f2, xyz[:, :M2]], axis=-1)
    wa3, wd3, gs3, bb3 = prep(W3, g3, bt3, 67)
    G3, CQ3 = _proj(faug3, wa3, wd3, M3)
    f3 = _sc_gather_max(i3.reshape(-1), G3.reshape(B * M2, 128),
                        CQ3.reshape(B * M3, 128), gs3, bb3, qc=32)
    f3 = f3.reshape(B, M3, 128)
    ofc = _fc(f3, jnp.transpose(Wfc), bfc[None, :])

    x3 = xyz[:, :M3]
    out_feat = jnp.transpose(ofc, (0, 2, 1))
    f3_t = jnp.transpose(f3, (0, 2, 1))
    idx_out = jnp.broadcast_to(
        jnp.arange(M3, dtype=jnp.int64)[None, :], (B, M3)).astype(jnp.int64)
    return (x3, out_feat, idx_out, f3_t)


# speedup vs baseline: 1.1475x; 1.0676x over previous
import functools

import jax
import jax.numpy as jnp
from jax import lax
from jax.experimental import pallas as pl
from jax.experimental.pallas import tpu as pltpu
from jax.experimental.pallas import tpu_sc as plsc

_EPS = 1e-5
_BIG = 3e38
_K = 16
_NC, _NS = 2, 16
_NW = _NC * _NS
_GW = 128


def _topk_body(nq_blk, n_cand, xT_ref, qT_ref, idx_ref):
    b = pl.program_id(0)
    xT = xT_ref[0]
    qT = qT_ref[0]
    dot = lax.dot_general(qT, xT, (((0,), (0,)), ((), ())),
                          preferred_element_type=jnp.float32)
    x2 = jnp.sum(xT * xT, axis=0)
    dmat = x2[None, :] - (dot + dot)
    iota = lax.broadcasted_iota(jnp.int32, (nq_blk, n_cand), 1)
    cols = []
    for _ in range(_K):
        first = jnp.argmin(dmat, axis=1).astype(jnp.int32)[:, None]
        cols.append(first)
        dmat = jnp.where(iota == first, _BIG, dmat)
    idx_ref[0] = jnp.concatenate(cols, axis=1) + b * n_cand


def _topk(xT, qT, nq_blk=256):
    B, _, n = xT.shape
    M = qT.shape[2]
    return pl.pallas_call(
        functools.partial(_topk_body, nq_blk, n),
        grid=(B, M // nq_blk),
        in_specs=[
            pl.BlockSpec((1, 3, n), lambda b, i: (b, 0, 0)),
            pl.BlockSpec((1, 3, nq_blk), lambda b, i: (b, 0, i)),
        ],
        out_specs=pl.BlockSpec((1, nq_blk, _K), lambda b, i: (b, i, 0)),
        out_shape=jax.ShapeDtypeStruct((B, M, _K), jnp.int32),
    )(xT, qT)


def _proj_body(M, f_ref, x_ref, waf_ref, wax_ref, wdf_ref, wdx_ref,
               g_ref, cq_ref):
    f = f_ref[0]
    x = x_ref[0]
    g_ref[0] = (jnp.dot(f, waf_ref[...], preferred_element_type=jnp.float32)
                + jnp.dot(x, wax_ref[...], preferred_element_type=jnp.float32))
    cq_ref[0] = (jnp.dot(f[:M], wdf_ref[...],
                         preferred_element_type=jnp.float32)
                 + jnp.dot(x[:M], wdx_ref[...],
                           preferred_element_type=jnp.float32))


def _proj(feat, xyzc, wa, wd, M):
    B, n, C0 = feat.shape
    Cg = wa.shape[1]
    Cq = wd.shape[1]
    waf, wax = wa[:C0], wa[C0:]
    wdf, wdx = wd[:C0], wd[C0:]
    return pl.pallas_call(
        functools.partial(_proj_body, M),
        grid=(B,),
        in_specs=[
            pl.BlockSpec((1, n, C0), lambda b: (b, 0, 0)),
            pl.BlockSpec((1, n, 3), lambda b: (b, 0, 0)),
            pl.BlockSpec((C0, Cg), lambda b: (0, 0)),
            pl.BlockSpec((3, Cg), lambda b: (0, 0)),
            pl.BlockSpec((C0, Cq), lambda b: (0, 0)),
            pl.BlockSpec((3, Cq), lambda b: (0, 0)),
        ],
        out_specs=[
            pl.BlockSpec((1, n, Cg), lambda b: (b, 0, 0)),
            pl.BlockSpec((1, M, Cq), lambda b: (b, 0, 0)),
        ],
        out_shape=[
            jax.ShapeDtypeStruct((B, n, Cg), jnp.float32),
            jax.ShapeDtypeStruct((B, M, Cq), jnp.float32),
        ],
    )(feat, xyzc, waf, wax, wdf, wdx)


def _fc_body(f_ref, w_ref, b_ref, o_ref):
    o_ref[0] = jnp.dot(f_ref[0], w_ref[...],
                       preferred_element_type=jnp.float32) + b_ref[...]


def _fc(f3, wfc, bfc):
    B, M, C = f3.shape
    Cfc = wfc.shape[1]
    return pl.pallas_call(
        _fc_body,
        grid=(B,),
        in_specs=[
            pl.BlockSpec((1, M, C), lambda b: (b, 0, 0)),
            pl.BlockSpec((C, Cfc), lambda b: (0, 0)),
            pl.BlockSpec((1, Cfc), lambda b: (0, 0)),
        ],
        out_specs=pl.BlockSpec((1, M, Cfc), lambda b: (b, 0, 0)),
        out_shape=jax.ShapeDtypeStruct((B, M, Cfc), jnp.float32),
    )(f3, wfc, bfc)


def _sc_gather_max(idx_flat, table, cq, gs, bb, qc):
    R, Du = cq.shape
    D = table.shape[1]
    rw = R // _NW
    nchunks = rw // qc
    ngath = (qc * _K) // _GW
    mesh = plsc.VectorSubcoreMesh(core_axis_name="c", subcore_axis_name="s")

    @functools.partial(
        pl.kernel, mesh=mesh,
        out_type=jax.ShapeDtypeStruct((R, Du), jnp.float32),
        scratch_types=[
            pltpu.VMEM((qc * _K,), jnp.int32),
            pltpu.VMEM((qc * _K, D), jnp.float32),
            pltpu.VMEM((qc, Du), jnp.float32),
            pltpu.VMEM((qc, Du), jnp.float32),
            pltpu.VMEM((Du,), jnp.float32),
            pltpu.VMEM((Du,), jnp.float32),
            pltpu.SemaphoreType.DMA,
        ],
    )
    def k(idx_hbm, table_hbm, cq_hbm, gs_hbm, bb_hbm, out_hbm,
          idx_v, rows_v, cq_v, out_v, gs_v, bb_v, sem):
        wid = lax.axis_index("s") * _NC + lax.axis_index("c")
        pltpu.sync_copy(gs_hbm, gs_v)
        pltpu.sync_copy(bb_hbm, bb_v)
        for t in range(nchunks):
            r0 = wid * rw + t * qc
            pltpu.sync_copy(idx_hbm.at[pl.ds(r0 * _K, qc * _K)], idx_v)
            handles = []
            for g in range(ngath):
                handles.append(pltpu.async_copy(
                    table_hbm.at[idx_v.at[pl.ds(g * _GW, _GW)]],
                    rows_v.at[pl.ds(g * _GW, _GW)], sem))
            pltpu.sync_copy(cq_hbm.at[pl.ds(r0, qc)], cq_v)
            for h in handles:
                h.wait()

            def qbody(q, carry):
                for c in range(Du // 16):
                    sl = pl.ds(c * 16, 16)
                    acc = rows_v[q * _K, sl]
                    for j in range(1, _K):
                        acc = jnp.maximum(acc, rows_v[q * _K + j, sl])
                    val = (acc + cq_v[q, sl]) * gs_v[sl] + bb_v[sl]
                    out_v[q, sl] = jnp.maximum(val, 0.0)
                return carry

            lax.fori_loop(0, qc, qbody, jnp.int32(0))
            pltpu.sync_copy(out_v, out_hbm.at[pl.ds(r0, qc)])

    return k(idx_flat, table, cq, gs, bb)


def kernel(pcd, W1, g1, bt1, W2, g2, bt2, W3, g3, bt3, Wfc, bfc):
    B, N, _ = pcd.shape
    M1, M2, M3 = N // 2, N // 4, N // 8
    s = (1.0 + _EPS) ** -0.5

    xyz = pcd[..., 0:3]
    xyzT = jnp.transpose(xyz, (0, 2, 1))

    def prep(W, g, b, C):
        wa = jnp.transpose(W[:, :C])
        wd = jnp.transpose(W[:, C:] - W[:, :C])
        if wa.shape[1] < 128:
            wa = jnp.pad(wa, ((0, 0), (0, 128 - wa.shape[1])))
        return wa, wd, g * s, b

    i1 = _topk(xyzT, xyzT[:, :, :M1])
    i2 = _topk(xyzT[:, :, :M1], xyzT[:, :, :M2])
    i3 = _topk(xyzT[:, :, :M2], xyzT[:, :, :M3])

    wa1, wd1, gs1, bb1 = prep(W1, g1, bt1, 6)
    G1, CQ1 = _proj(pcd[..., 3:6], xyz, wa1, wd1, M1)
    f1 = _sc_gather_max(i1.reshape(-1), G1.reshape(B * N, 128),
                        CQ1.reshape(B * M1, 64), gs1, bb1, qc=32)
    f1 = f1.reshape(B, M1, 64)

    wa2, wd2, gs2, bb2 = prep(W2, g2, bt2, 67)
    G2, CQ2 = _proj(f1, xyz[:, :M1], wa2, wd2, M2)
    f2 = _sc_gather_max(i2.reshape(-1), G2.reshape(B * M1, 128),
                        CQ2.reshape(B * M2, 64), gs2, bb2, qc=32)
    f2 = f2.reshape(B, M2, 64)

    wa3, wd3, gs3, bb3 = prep(W3, g3, bt3, 67)
    G3, CQ3 = _proj(f2, xyz[:, :M2], wa3, wd3, M3)
    f3 = _sc_gather_max(i3.reshape(-1), G3.reshape(B * M2, 128),
                        CQ3.reshape(B * M3, 128), gs3, bb3, qc=32)
    f3 = f3.reshape(B, M3, 128)
    ofc = _fc(f3, jnp.transpose(Wfc), bfc[None, :])

    x3 = xyz[:, :M3]
    out_feat = jnp.transpose(ofc, (0, 2, 1))
    f3_t = jnp.transpose(f3, (0, 2, 1))
    idx_out = jnp.broadcast_to(
        jnp.arange(M3, dtype=jnp.int64)[None, :], (B, M3)).astype(jnp.int64)
    return (x3, out_feat, idx_out, f3_t)
